# baseline (speedup 1.0000x reference)
import functools

import jax
import jax.numpy as jnp
from jax import lax
from jax.experimental import pallas as pl
from jax.experimental.pallas import tpu as pltpu
from jax.experimental.pallas import tpu_sc as plsc

_N = 10000
_D = 128
_E = 320000
_NC = 2
_NS = 16
_NW = _NC * _NS
_K = 128
_CPW = 80
_MB = 16
_NB = _CPW // _MB
_EPAD = _NW * _CPW * _K
_RPS = 624
_RTAIL = _N - _RPS * _NS


def _pack_bf16_pairs(v):
    n = v.shape[0]
    v4 = v.reshape(n, _D // 32, 2, 16).astype(jnp.bfloat16)
    bits = lax.bitcast_convert_type(v4, jnp.uint16).astype(jnp.int32)
    packed = (bits[:, :, 1, :] << 16) | bits[:, :, 0, :]
    return packed.reshape(n, _D // 2)


def _layernorm(x):
    def body(x_ref, o_ref):
        v = x_ref[...]
        m = jnp.mean(v, axis=-1, keepdims=True)
        d = v - m
        var = jnp.mean(d * d, axis=-1, keepdims=True)
        o_ref[...] = _pack_bf16_pairs(d * lax.rsqrt(var + 1e-5))

    return pl.pallas_call(
        body,
        out_shape=jax.ShapeDtypeStruct((_N, _D // 2), jnp.int32),
        grid=(10,),
        in_specs=[pl.BlockSpec((_N // 10, _D), lambda i: (i, 0))],
        out_specs=pl.BlockSpec((_N // 10, _D // 2), lambda i: (i, 0)),
    )(x)


def _combine(a, b):
    def body(a_ref, b_ref, o_ref, obf_ref):
        v = a_ref[...] + b_ref[...]
        o_ref[...] = v
        obf_ref[...] = _pack_bf16_pairs(v)

    return pl.pallas_call(
        body,
        out_shape=(
            jax.ShapeDtypeStruct((_N, _D), jnp.float32),
            jax.ShapeDtypeStruct((_N, _D // 2), jnp.int32),
        ),
        grid=(10,),
        in_specs=[
            pl.BlockSpec((_N // 10, _D), lambda i: (i, 0)),
            pl.BlockSpec((_N // 10, _D), lambda i: (i, 0)),
        ],
        out_specs=(
            pl.BlockSpec((_N // 10, _D), lambda i: (i, 0)),
            pl.BlockSpec((_N // 10, _D // 2), lambda i: (i, 0)),
        ),
    )(a, b)


def _add2(a, b):
    def body(a_ref, b_ref, o_ref):
        o_ref[...] = a_ref[...] + b_ref[...]

    return pl.pallas_call(
        body,
        out_shape=jax.ShapeDtypeStruct((_N, _D), jnp.float32),
        grid=(10,),
        in_specs=[
            pl.BlockSpec((_N // 10, _D), lambda i: (i, 0)),
            pl.BlockSpec((_N // 10, _D), lambda i: (i, 0)),
        ],
        out_specs=pl.BlockSpec((_N // 10, _D), lambda i: (i, 0)),
    )(a, b)


def _spmm_body(src_h, dst_h, w_h, x_h, init_h, out_h,
               src_v, dst_v, w_v, rows_a, rows_b, rows_f, acc,
               sem_a, sem_b):
    c = lax.axis_index("c")
    s = lax.axis_index("s")
    wid = c * _NS + s
    r0 = s * _RPS
    pltpu.sync_copy(init_h.at[c, pl.ds(r0, _RPS)], acc.at[pl.ds(r0, _RPS)])

    @pl.when(s == 0)
    def _():
        pltpu.sync_copy(init_h.at[c, pl.ds(_RPS * _NS, _RTAIL)],
                        acc.at[pl.ds(_RPS * _NS, _RTAIL)])

    plsc.subcore_barrier()

    def gather_start(g, rows, sem):
        pltpu.async_copy(x_h.at[src_v.at[g]], rows, sem)

    def gather_wait(g, rows, sem):
        pltpu.make_async_copy(x_h.at[src_v.at[g]], rows, sem).wait()

    def scale_scatter(g, rows_bf):
        def scale_body(g16, carry):
            wg = w_v[g, pl.ds(g16 * 16, 16)]
            for e16 in range(16):
                e = g16 * 16 + e16
                wb = wg.at[jnp.full((16,), e16, jnp.int32)].get(
                    mode="promise_in_bounds")
                for r in range(_D // 32):
                    u = rows_bf[e, pl.ds(r * 16, 16)]
                    lo = lax.bitcast_convert_type(u << 16, jnp.float32)
                    hi = lax.bitcast_convert_type(u & jnp.int32(-65536),
                                                  jnp.float32)
                    rows_f[e, pl.ds(r * 32, 16)] = lo * wb
                    rows_f[e, pl.ds(r * 32 + 16, 16)] = hi * wb
            return carry

        lax.fori_loop(0, _K // 16, scale_body, 0)
        pltpu.sync_copy(rows_f, acc.at[dst_v.at[g]], add=True)

    def blk_body(b, carry):
        c0 = b * _MB
        pltpu.sync_copy(src_h.at[wid, pl.ds(c0, _MB)], src_v)
        pltpu.sync_copy(dst_h.at[wid, pl.ds(c0, _MB)], dst_v)
        pltpu.sync_copy(w_h.at[wid, pl.ds(c0, _MB)], w_v)
        gather_start(0, rows_a, sem_a)

        def pair_body(i, carry2):
            g = i * 2
            gather_wait(g, rows_a, sem_a)
            gather_start(g + 1, rows_b, sem_b)
            scale_scatter(g, rows_a)
            gather_wait(g + 1, rows_b, sem_b)

            @pl.when(g + 2 < _MB)
            def _():
                gather_start(g + 2, rows_a, sem_a)

            scale_scatter(g + 1, rows_b)
            return carry2

        lax.fori_loop(0, _MB // 2, pair_body, 0)
        return carry

    lax.fori_loop(0, _NB, blk_body, 0)
    plsc.subcore_barrier()
    pltpu.sync_copy(acc.at[pl.ds(r0, _RPS)], out_h.at[c, pl.ds(r0, _RPS)])

    @pl.when(s == 0)
    def _():
        pltpu.sync_copy(acc.at[pl.ds(_RPS * _NS, _RTAIL)],
                        out_h.at[c, pl.ds(_RPS * _NS, _RTAIL)])


def _spmm_sc(src, dst, w, x, init):
    mesh = plsc.VectorSubcoreMesh(core_axis_name="c", subcore_axis_name="s")
    f = functools.partial(
        pl.kernel,
        out_type=jax.ShapeDtypeStruct((_NC, _N, _D), jnp.float32),
        mesh=mesh,
        scratch_types=[
            pltpu.VMEM((_MB, _K), jnp.int32),
            pltpu.VMEM((_MB, _K), jnp.int32),
            pltpu.VMEM((_MB, _K), jnp.float32),
            pltpu.VMEM((_K, _D // 2), jnp.int32),
            pltpu.VMEM((_K, _D // 2), jnp.int32),
            pltpu.VMEM((_K, _D), jnp.float32),
            pltpu.VMEM_SHARED((_N, _D), jnp.float32),
            pltpu.SemaphoreType.DMA,
            pltpu.SemaphoreType.DMA,
        ],
        compiler_params=pltpu.CompilerParams(use_tc_tiling_on_sc=False),
    )(_spmm_body)
    return f(src, dst, w, x, init)


def kernel(edge_index, edge_weight, embeds):
    src = edge_index[1]
    dst = edge_index[0]
    pad = _EPAD - _E
    spread = (jnp.arange(pad, dtype=jnp.int32) * 13) % _N
    src_p = jnp.concatenate([src, spread]).reshape(_NW, _CPW, _K)
    dst_p = jnp.concatenate([dst, spread]).reshape(_NW, _CPW, _K)
    w_p = jnp.pad(edge_weight, (0, pad)).reshape(_NW, _CPW, _K)

    x = _layernorm(embeds)
    zeros2 = jnp.zeros((_NC, _N, _D), jnp.float32)
    p = _spmm_sc(src_p, dst_p, w_p, x, zeros2)
    y1, y1bf = _combine(p[0], p[1])
    init2 = jnp.concatenate([y1[None], jnp.zeros((1, _N, _D), jnp.float32)], axis=0)
    q = _spmm_sc(src_p, dst_p, w_p, y1bf, init2)
    return _add2(q[0], q[1])


# speedup vs baseline: 2.1686x; 2.1686x over previous
import functools

import jax
import jax.numpy as jnp
from jax import lax
from jax.experimental import pallas as pl
from jax.experimental.pallas import tpu as pltpu
from jax.experimental.pallas import tpu_sc as plsc

_N = 10000
_D = 128
_E = 320000
_NC = 2
_NS = 16
_NW = _NC * _NS
_K = 128
_CPW = 80
_MB = 16
_NB = _CPW // _MB
_EPAD = _NW * _CPW * _K
_RPS = 624
_RTAIL = _N - _RPS * _NS


def _layernorm(x):
    def body(x_ref, o_ref):
        v = x_ref[...]
        m = jnp.mean(v, axis=-1, keepdims=True)
        d = v - m
        var = jnp.mean(d * d, axis=-1, keepdims=True)
        o_ref[...] = d * lax.rsqrt(var + 1e-5)

    return pl.pallas_call(
        body,
        out_shape=jax.ShapeDtypeStruct((_N, _D), jnp.float32),
        grid=(10,),
        in_specs=[pl.BlockSpec((_N // 10, _D), lambda i: (i, 0))],
        out_specs=pl.BlockSpec((_N // 10, _D), lambda i: (i, 0)),
    )(x)


def _add2(a, b):
    def body(a_ref, b_ref, o_ref):
        o_ref[...] = a_ref[...] + b_ref[...]

    return pl.pallas_call(
        body,
        out_shape=jax.ShapeDtypeStruct((_N, _D), jnp.float32),
        grid=(10,),
        in_specs=[
            pl.BlockSpec((_N // 10, _D), lambda i: (i, 0)),
            pl.BlockSpec((_N // 10, _D), lambda i: (i, 0)),
        ],
        out_specs=pl.BlockSpec((_N // 10, _D), lambda i: (i, 0)),
    )(a, b)


def _spmm_body(src_h, dst_h, w_h, x_h, init_h, out_h,
               src_v, dst_v, w_v, rows_a, rows_b, acc, sem_a, sem_b):
    c = lax.axis_index("c")
    s = lax.axis_index("s")
    wid = c * _NS + s
    r0 = s * _RPS
    pltpu.sync_copy(init_h.at[c, pl.ds(r0, _RPS)], acc.at[pl.ds(r0, _RPS)])

    @pl.when(s == 0)
    def _():
        pltpu.sync_copy(init_h.at[c, pl.ds(_RPS * _NS, _RTAIL)],
                        acc.at[pl.ds(_RPS * _NS, _RTAIL)])

    plsc.subcore_barrier()

    def gather_start(g, rows, sem):
        pltpu.async_copy(x_h.at[src_v.at[g]], rows, sem)

    def gather_wait(g, rows, sem):
        pltpu.make_async_copy(x_h.at[src_v.at[g]], rows, sem).wait()

    def scale_scatter(g, rows):
        def scale_body(g16, carry):
            wg = w_v[g, pl.ds(g16 * 16, 16)]
            for e16 in range(16):
                e = g16 * 16 + e16
                wb = wg.at[jnp.full((16,), e16, jnp.int32)].get(
                    mode="promise_in_bounds")
                for r in range(_D // 16):
                    rows[e, pl.ds(r * 16, 16)] = rows[e, pl.ds(r * 16, 16)] * wb
            return carry

        lax.fori_loop(0, _K // 16, scale_body, 0)
        pltpu.sync_copy(rows, acc.at[dst_v.at[g]], add=True)

    def blk_body(b, carry):
        c0 = b * _MB
        pltpu.sync_copy(src_h.at[wid, pl.ds(c0, _MB)], src_v)
        pltpu.sync_copy(dst_h.at[wid, pl.ds(c0, _MB)], dst_v)
        pltpu.sync_copy(w_h.at[wid, pl.ds(c0, _MB)], w_v)
        gather_start(0, rows_a, sem_a)

        def pair_body(i, carry2):
            g = i * 2
            gather_wait(g, rows_a, sem_a)
            gather_start(g + 1, rows_b, sem_b)
            scale_scatter(g, rows_a)
            gather_wait(g + 1, rows_b, sem_b)

            @pl.when(g + 2 < _MB)
            def _():
                gather_start(g + 2, rows_a, sem_a)

            scale_scatter(g + 1, rows_b)
            return carry2

        lax.fori_loop(0, _MB // 2, pair_body, 0)
        return carry

    lax.fori_loop(0, _NB, blk_body, 0)
    plsc.subcore_barrier()
    pltpu.sync_copy(acc.at[pl.ds(r0, _RPS)], out_h.at[c, pl.ds(r0, _RPS)])

    @pl.when(s == 0)
    def _():
        pltpu.sync_copy(acc.at[pl.ds(_RPS * _NS, _RTAIL)],
                        out_h.at[c, pl.ds(_RPS * _NS, _RTAIL)])


def _spmm_sc(src, dst, w, x, init):
    mesh = plsc.VectorSubcoreMesh(core_axis_name="c", subcore_axis_name="s")
    f = functools.partial(
        pl.kernel,
        out_type=jax.ShapeDtypeStruct((_NC, _N, _D), jnp.float32),
        mesh=mesh,
        scratch_types=[
            pltpu.VMEM((_MB, _K), jnp.int32),
            pltpu.VMEM((_MB, _K), jnp.int32),
            pltpu.VMEM((_MB, _K), jnp.float32),
            pltpu.VMEM((_K, _D), jnp.float32),
            pltpu.VMEM((_K, _D), jnp.float32),
            pltpu.VMEM_SHARED((_N, _D), jnp.float32),
            pltpu.SemaphoreType.DMA,
            pltpu.SemaphoreType.DMA,
        ],
    )(_spmm_body)
    return f(src, dst, w, x, init)


def kernel(edge_index, edge_weight, embeds):
    src = edge_index[1]
    dst = edge_index[0]
    pad = _EPAD - _E
    spread = (jnp.arange(pad, dtype=jnp.int32) * 13) % _N
    src_p = jnp.concatenate([src, spread]).reshape(_NW, _CPW, _K)
    dst_p = jnp.concatenate([dst, spread]).reshape(_NW, _CPW, _K)
    w_p = jnp.pad(edge_weight, (0, pad)).reshape(_NW, _CPW, _K)

    x = _layernorm(embeds)
    zeros2 = jnp.zeros((_NC, _N, _D), jnp.float32)
    p = _spmm_sc(src_p, dst_p, w_p, x, zeros2)
    y1 = _add2(p[0], p[1])
    init2 = jnp.concatenate([y1[None], jnp.zeros((1, _N, _D), jnp.float32)], axis=0)
    q = _spmm_sc(src_p, dst_p, w_p, y1, init2)
    return _add2(q[0], q[1])


# speedup vs baseline: 2.1961x; 1.0127x over previous
import functools

import jax
import jax.numpy as jnp
from jax import lax
from jax.experimental import pallas as pl
from jax.experimental.pallas import tpu as pltpu
from jax.experimental.pallas import tpu_sc as plsc

_N = 10000
_D = 128
_E = 320000
_NC = 2
_NS = 16
_NW = _NC * _NS
_K = 128
_CPW = 80
_MB = 16
_NB = _CPW // _MB
_EPAD = _NW * _CPW * _K
_RPS = 624
_RTAIL = _N - _RPS * _NS


def _layernorm(x):
    def body(x_ref, o_ref):
        v = x_ref[...]
        m = jnp.mean(v, axis=-1, keepdims=True)
        d = v - m
        var = jnp.mean(d * d, axis=-1, keepdims=True)
        o_ref[...] = d * lax.rsqrt(var + 1e-5)

    return pl.pallas_call(
        body,
        out_shape=jax.ShapeDtypeStruct((_N, _D), jnp.float32),
        grid=(10,),
        in_specs=[pl.BlockSpec((_N // 10, _D), lambda i: (i, 0))],
        out_specs=pl.BlockSpec((_N // 10, _D), lambda i: (i, 0)),
    )(x)


def _add2(a, b):
    def body(a_ref, b_ref, o_ref):
        o_ref[...] = a_ref[...] + b_ref[...]

    return pl.pallas_call(
        body,
        out_shape=jax.ShapeDtypeStruct((_N, _D), jnp.float32),
        grid=(10,),
        in_specs=[
            pl.BlockSpec((_N // 10, _D), lambda i: (i, 0)),
            pl.BlockSpec((_N // 10, _D), lambda i: (i, 0)),
        ],
        out_specs=pl.BlockSpec((_N // 10, _D), lambda i: (i, 0)),
    )(a, b)


def _spmm_body(src_h, dst_h, w_h, x_h, init_h, out_h,
               src_v, dst_v, w_v, rows_a, rows_b, acc, sem_a, sem_b):
    c = lax.axis_index("c")
    s = lax.axis_index("s")
    wid = c * _NS + s
    r0 = s * _RPS
    pltpu.sync_copy(init_h.at[c, pl.ds(r0, _RPS)], acc.at[pl.ds(r0, _RPS)])

    @pl.when(s == 0)
    def _():
        pltpu.sync_copy(init_h.at[c, pl.ds(_RPS * _NS, _RTAIL)],
                        acc.at[pl.ds(_RPS * _NS, _RTAIL)])

    plsc.subcore_barrier()

    def gather_start(g, rows, sem):
        pltpu.async_copy(x_h.at[src_v.at[g]], rows, sem)

    def gather_wait(g, rows, sem):
        pltpu.make_async_copy(x_h.at[src_v.at[g]], rows, sem).wait()

    def scale_scatter(g, rows):
        def scale_body(g16, carry):
            wg = w_v[g, pl.ds(g16 * 16, 16)]
            for e16 in range(16):
                e = g16 * 16 + e16
                wb = wg.at[jnp.full((16,), e16, jnp.int32)].get(
                    mode="promise_in_bounds")
                for r in range(_D // 16):
                    rows[e, pl.ds(r * 16, 16)] = rows[e, pl.ds(r * 16, 16)] * wb
            return carry

        lax.fori_loop(0, _K // 16, scale_body, 0)
        pltpu.sync_copy(rows, acc.at[dst_v.at[g]], add=True)

    def blk_body(b, carry):
        c0 = b * _MB
        pltpu.sync_copy(src_h.at[wid, pl.ds(c0, _MB)], src_v)
        pltpu.sync_copy(dst_h.at[wid, pl.ds(c0, _MB)], dst_v)
        pltpu.sync_copy(w_h.at[wid, pl.ds(c0, _MB)], w_v)
        gather_start(0, rows_a, sem_a)

        def pair_body(i, carry2):
            g = i * 2
            gather_wait(g, rows_a, sem_a)
            gather_start(g + 1, rows_b, sem_b)
            scale_scatter(g, rows_a)

            @pl.when(g + 2 < _MB)
            def _():
                gather_start(g + 2, rows_a, sem_a)

            gather_wait(g + 1, rows_b, sem_b)
            scale_scatter(g + 1, rows_b)
            return carry2

        lax.fori_loop(0, _MB // 2, pair_body, 0)
        return carry

    lax.fori_loop(0, _NB, blk_body, 0)
    plsc.subcore_barrier()
    pltpu.sync_copy(acc.at[pl.ds(r0, _RPS)], out_h.at[c, pl.ds(r0, _RPS)])

    @pl.when(s == 0)
    def _():
        pltpu.sync_copy(acc.at[pl.ds(_RPS * _NS, _RTAIL)],
                        out_h.at[c, pl.ds(_RPS * _NS, _RTAIL)])


def _spmm_sc(src, dst, w, x, init):
    mesh = plsc.VectorSubcoreMesh(core_axis_name="c", subcore_axis_name="s")
    f = functools.partial(
        pl.kernel,
        out_type=jax.ShapeDtypeStruct((_NC, _N, _D), jnp.float32),
        mesh=mesh,
        scratch_types=[
            pltpu.VMEM((_MB, _K), jnp.int32),
            pltpu.VMEM((_MB, _K), jnp.int32),
            pltpu.VMEM((_MB, _K), jnp.float32),
            pltpu.VMEM((_K, _D), jnp.float32),
            pltpu.VMEM((_K, _D), jnp.float32),
            pltpu.VMEM_SHARED((_N, _D), jnp.float32),
            pltpu.SemaphoreType.DMA,
            pltpu.SemaphoreType.DMA,
        ],
    )(_spmm_body)
    return f(src, dst, w, x, init)


def kernel(edge_index, edge_weight, embeds):
    src = edge_index[1]
    dst = edge_index[0]
    pad = _EPAD - _E
    spread = (jnp.arange(pad, dtype=jnp.int32) * 13) % _N
    src_p = jnp.concatenate([src, spread]).reshape(_NW, _CPW, _K)
    dst_p = jnp.concatenate([dst, spread]).reshape(_NW, _CPW, _K)
    w_p = jnp.pad(edge_weight, (0, pad)).reshape(_NW, _CPW, _K)

    x = _layernorm(embeds)
    zeros2 = jnp.zeros((_NC, _N, _D), jnp.float32)
    p = _spmm_sc(src_p, dst_p, w_p, x, zeros2)
    y1 = _add2(p[0], p[1])
    init2 = jnp.concatenate([y1[None], jnp.zeros((1, _N, _D), jnp.float32)], axis=0)
    q = _spmm_sc(src_p, dst_p, w_p, y1, init2)
    return _add2(q[0], q[1])


# speedup vs baseline: 2.2407x; 1.0203x over previous
import functools

import jax
import jax.numpy as jnp
from jax import lax
from jax.experimental import pallas as pl
from jax.experimental.pallas import tpu as pltpu
from jax.experimental.pallas import tpu_sc as plsc

_N = 10000
_D = 128
_E = 320000
_NC = 2
_NS = 16
_NW = _NC * _NS
_K = 128
_CPW = 80
_MB = 16
_NB = _CPW // _MB
_EPAD = _NW * _CPW * _K
_RPS = 624
_RTAIL = _N - _RPS * _NS


def _layernorm(x):
    def body(x_ref, o_ref):
        v = x_ref[...]
        m = jnp.mean(v, axis=-1, keepdims=True)
        d = v - m
        var = jnp.mean(d * d, axis=-1, keepdims=True)
        o_ref[...] = d * lax.rsqrt(var + 1e-5)

    return pl.pallas_call(
        body,
        out_shape=jax.ShapeDtypeStruct((_N, _D), jnp.float32),
        grid=(10,),
        in_specs=[pl.BlockSpec((_N // 10, _D), lambda i: (i, 0))],
        out_specs=pl.BlockSpec((_N // 10, _D), lambda i: (i, 0)),
    )(x)


def _add2(a, b):
    def body(a_ref, b_ref, o_ref):
        o_ref[...] = a_ref[...] + b_ref[...]

    return pl.pallas_call(
        body,
        out_shape=jax.ShapeDtypeStruct((_N, _D), jnp.float32),
        grid=(10,),
        in_specs=[
            pl.BlockSpec((_N // 10, _D), lambda i: (i, 0)),
            pl.BlockSpec((_N // 10, _D), lambda i: (i, 0)),
        ],
        out_specs=pl.BlockSpec((_N // 10, _D), lambda i: (i, 0)),
    )(a, b)


def _spmm_body(src_h, dst_h, w_h, x_h, init0_h, init1_h, out_h,
               src_v, dst_v, w_v, rows_a, rows_b, acc, sem_a, sem_b):
    c = lax.axis_index("c")
    s = lax.axis_index("s")
    wid = c * _NS + s
    r0 = s * _RPS
    @pl.when(c == 0)
    def _():
        pltpu.sync_copy(init0_h.at[pl.ds(r0, _RPS)], acc.at[pl.ds(r0, _RPS)])

        @pl.when(s == 0)
        def _():
            pltpu.sync_copy(init0_h.at[pl.ds(_RPS * _NS, _RTAIL)],
                            acc.at[pl.ds(_RPS * _NS, _RTAIL)])

    @pl.when(c == 1)
    def _():
        pltpu.sync_copy(init1_h.at[pl.ds(r0, _RPS)], acc.at[pl.ds(r0, _RPS)])

        @pl.when(s == 0)
        def _():
            pltpu.sync_copy(init1_h.at[pl.ds(_RPS * _NS, _RTAIL)],
                            acc.at[pl.ds(_RPS * _NS, _RTAIL)])

    plsc.subcore_barrier()

    def gather_start(g, rows, sem):
        pltpu.async_copy(x_h.at[src_v.at[g]], rows, sem)

    def gather_wait(g, rows, sem):
        pltpu.make_async_copy(x_h.at[src_v.at[g]], rows, sem).wait()

    def scale_scatter(g, rows):
        def scale_body(g16, carry):
            wg = w_v[g, pl.ds(g16 * 16, 16)]
            for e16 in range(16):
                e = g16 * 16 + e16
                wb = wg.at[jnp.full((16,), e16, jnp.int32)].get(
                    mode="promise_in_bounds")
                for r in range(_D // 16):
                    rows[e, pl.ds(r * 16, 16)] = rows[e, pl.ds(r * 16, 16)] * wb
            return carry

        lax.fori_loop(0, _K // 16, scale_body, 0)
        pltpu.sync_copy(rows, acc.at[dst_v.at[g]], add=True)

    def blk_body(b, carry):
        c0 = b * _MB
        pltpu.sync_copy(src_h.at[wid, pl.ds(c0, _MB)], src_v)
        pltpu.sync_copy(dst_h.at[wid, pl.ds(c0, _MB)], dst_v)
        pltpu.sync_copy(w_h.at[wid, pl.ds(c0, _MB)], w_v)
        gather_start(0, rows_a, sem_a)

        def pair_body(i, carry2):
            g = i * 2
            gather_wait(g, rows_a, sem_a)
            gather_start(g + 1, rows_b, sem_b)
            scale_scatter(g, rows_a)

            @pl.when(g + 2 < _MB)
            def _():
                gather_start(g + 2, rows_a, sem_a)

            gather_wait(g + 1, rows_b, sem_b)
            scale_scatter(g + 1, rows_b)
            return carry2

        lax.fori_loop(0, _MB // 2, pair_body, 0)
        return carry

    lax.fori_loop(0, _NB, blk_body, 0)
    plsc.subcore_barrier()
    pltpu.sync_copy(acc.at[pl.ds(r0, _RPS)], out_h.at[c, pl.ds(r0, _RPS)])

    @pl.when(s == 0)
    def _():
        pltpu.sync_copy(acc.at[pl.ds(_RPS * _NS, _RTAIL)],
                        out_h.at[c, pl.ds(_RPS * _NS, _RTAIL)])


def _spmm_sc(src, dst, w, x, init0, init1):
    mesh = plsc.VectorSubcoreMesh(core_axis_name="c", subcore_axis_name="s")
    f = functools.partial(
        pl.kernel,
        out_type=jax.ShapeDtypeStruct((_NC, _N, _D), jnp.float32),
        mesh=mesh,
        scratch_types=[
            pltpu.VMEM((_MB, _K), jnp.int32),
            pltpu.VMEM((_MB, _K), jnp.int32),
            pltpu.VMEM((_MB, _K), jnp.float32),
            pltpu.VMEM((_K, _D), jnp.float32),
            pltpu.VMEM((_K, _D), jnp.float32),
            pltpu.VMEM_SHARED((_N, _D), jnp.float32),
            pltpu.SemaphoreType.DMA,
            pltpu.SemaphoreType.DMA,
        ],
    )(_spmm_body)
    return f(src, dst, w, x, init0, init1)


def kernel(edge_index, edge_weight, embeds):
    src = edge_index[1]
    dst = edge_index[0]
    pad = _EPAD - _E
    spread = (jnp.arange(pad, dtype=jnp.int32) * 13) % _N
    src_p = jnp.concatenate([src, spread]).reshape(_NW, _CPW, _K)
    dst_p = jnp.concatenate([dst, spread]).reshape(_NW, _CPW, _K)
    w_p = jnp.pad(edge_weight, (0, pad)).reshape(_NW, _CPW, _K)

    x = _layernorm(embeds)
    zeros = jnp.zeros((_N, _D), jnp.float32)
    p = _spmm_sc(src_p, dst_p, w_p, x, zeros, zeros)
    y1 = _add2(p[0], p[1])
    q = _spmm_sc(src_p, dst_p, w_p, y1, y1, zeros)
    return _add2(q[0], q[1])


# speedup vs baseline: 2.3370x; 1.0430x over previous
import functools

import jax
import jax.numpy as jnp
from jax import lax
from jax.experimental import pallas as pl
from jax.experimental.pallas import tpu as pltpu
from jax.experimental.pallas import tpu_sc as plsc

_N = 10000
_D = 128
_E = 320000
_NC = 2
_NS = 16
_NW = _NC * _NS
_K = 128
_CPW = 80
_MB = 40
_NB = _CPW // _MB
_EPAD = _NW * _CPW * _K
_RPS = 624
_RTAIL = _N - _RPS * _NS


def _layernorm(x):
    def body(x_ref, o_ref):
        v = x_ref[...]
        m = jnp.mean(v, axis=-1, keepdims=True)
        d = v - m
        var = jnp.mean(d * d, axis=-1, keepdims=True)
        o_ref[...] = d * lax.rsqrt(var + 1e-5)

    return pl.pallas_call(
        body,
        out_shape=jax.ShapeDtypeStruct((_N, _D), jnp.float32),
        grid=(10,),
        in_specs=[pl.BlockSpec((_N // 10, _D), lambda i: (i, 0))],
        out_specs=pl.BlockSpec((_N // 10, _D), lambda i: (i, 0)),
    )(x)


def _add2(a, b):
    def body(a_ref, b_ref, o_ref):
        o_ref[...] = a_ref[...] + b_ref[...]

    return pl.pallas_call(
        body,
        out_shape=jax.ShapeDtypeStruct((_N, _D), jnp.float32),
        grid=(10,),
        in_specs=[
            pl.BlockSpec((_N // 10, _D), lambda i: (i, 0)),
            pl.BlockSpec((_N // 10, _D), lambda i: (i, 0)),
        ],
        out_specs=pl.BlockSpec((_N // 10, _D), lambda i: (i, 0)),
    )(a, b)


def _spmm_body(src_h, dst_h, w_h, x_h, init0_h, init1_h, out_h,
               src_v, dst_v, w_v, rows_a, rows_b, acc, sem_a, sem_b):
    c = lax.axis_index("c")
    s = lax.axis_index("s")
    wid = c * _NS + s
    r0 = s * _RPS
    @pl.when(c == 0)
    def _():
        pltpu.sync_copy(init0_h.at[pl.ds(r0, _RPS)], acc.at[pl.ds(r0, _RPS)])

        @pl.when(s == 0)
        def _():
            pltpu.sync_copy(init0_h.at[pl.ds(_RPS * _NS, _RTAIL)],
                            acc.at[pl.ds(_RPS * _NS, _RTAIL)])

    @pl.when(c == 1)
    def _():
        pltpu.sync_copy(init1_h.at[pl.ds(r0, _RPS)], acc.at[pl.ds(r0, _RPS)])

        @pl.when(s == 0)
        def _():
            pltpu.sync_copy(init1_h.at[pl.ds(_RPS * _NS, _RTAIL)],
                            acc.at[pl.ds(_RPS * _NS, _RTAIL)])

    plsc.subcore_barrier()

    def gather_start(g, rows, sem):
        pltpu.async_copy(x_h.at[src_v.at[g]], rows, sem)

    def gather_wait(g, rows, sem):
        pltpu.make_async_copy(x_h.at[src_v.at[g]], rows, sem).wait()

    def scale_scatter(g, rows):
        def scale_body(g16, carry):
            wg = w_v[g, pl.ds(g16 * 16, 16)]
            for e16 in range(16):
                e = g16 * 16 + e16
                wb = wg.at[jnp.full((16,), e16, jnp.int32)].get(
                    mode="promise_in_bounds")
                for r in range(_D // 16):
                    rows[e, pl.ds(r * 16, 16)] = rows[e, pl.ds(r * 16, 16)] * wb
            return carry

        lax.fori_loop(0, _K // 16, scale_body, 0)
        pltpu.sync_copy(rows, acc.at[dst_v.at[g]], add=True)

    def blk_body(b, carry):
        c0 = b * _MB
        pltpu.sync_copy(src_h.at[wid, pl.ds(c0, _MB)], src_v)
        pltpu.sync_copy(dst_h.at[wid, pl.ds(c0, _MB)], dst_v)
        pltpu.sync_copy(w_h.at[wid, pl.ds(c0, _MB)], w_v)
        gather_start(0, rows_a, sem_a)

        def pair_body(i, carry2):
            g = i * 2
            gather_wait(g, rows_a, sem_a)
            gather_start(g + 1, rows_b, sem_b)
            scale_scatter(g, rows_a)

            @pl.when(g + 2 < _MB)
            def _():
                gather_start(g + 2, rows_a, sem_a)

            gather_wait(g + 1, rows_b, sem_b)
            scale_scatter(g + 1, rows_b)
            return carry2

        lax.fori_loop(0, _MB // 2, pair_body, 0)
        return carry

    lax.fori_loop(0, _NB, blk_body, 0)
    plsc.subcore_barrier()
    pltpu.sync_copy(acc.at[pl.ds(r0, _RPS)], out_h.at[c, pl.ds(r0, _RPS)])

    @pl.when(s == 0)
    def _():
        pltpu.sync_copy(acc.at[pl.ds(_RPS * _NS, _RTAIL)],
                        out_h.at[c, pl.ds(_RPS * _NS, _RTAIL)])


def _spmm_sc(src, dst, w, x, init0, init1):
    mesh = plsc.VectorSubcoreMesh(core_axis_name="c", subcore_axis_name="s")
    f = functools.partial(
        pl.kernel,
        out_type=jax.ShapeDtypeStruct((_NC, _N, _D), jnp.float32),
        mesh=mesh,
        scratch_types=[
            pltpu.VMEM((_MB, _K), jnp.int32),
            pltpu.VMEM((_MB, _K), jnp.int32),
            pltpu.VMEM((_MB, _K), jnp.float32),
            pltpu.VMEM((_K, _D), jnp.float32),
            pltpu.VMEM((_K, _D), jnp.float32),
            pltpu.VMEM_SHARED((_N, _D), jnp.float32),
            pltpu.SemaphoreType.DMA,
            pltpu.SemaphoreType.DMA,
        ],
    )(_spmm_body)
    return f(src, dst, w, x, init0, init1)


def kernel(edge_index, edge_weight, embeds):
    src = edge_index[1]
    dst = edge_index[0]
    pad = _EPAD - _E
    spread = (jnp.arange(pad, dtype=jnp.int32) * 13) % _N
    src_p = jnp.concatenate([src, spread]).reshape(_NW, _CPW, _K)
    dst_p = jnp.concatenate([dst, spread]).reshape(_NW, _CPW, _K)
    w_p = jnp.pad(edge_weight, (0, pad)).reshape(_NW, _CPW, _K)

    x = _layernorm(embeds)
    zeros = jnp.zeros((_N, _D), jnp.float32)
    p = _spmm_sc(src_p, dst_p, w_p, x, zeros, zeros)
    y1 = _add2(p[0], p[1])
    q = _spmm_sc(src_p, dst_p, w_p, y1, y1, zeros)
    return _add2(q[0], q[1])


# speedup vs baseline: 2.3911x; 1.0231x over previous
import functools

import jax
import jax.numpy as jnp
from jax import lax
from jax.experimental import pallas as pl
from jax.experimental.pallas import tpu as pltpu
from jax.experimental.pallas import tpu_sc as plsc

_N = 10000
_D = 128
_E = 320000
_NC = 2
_NS = 16
_NW = _NC * _NS
_K = 128
_CPW = 80
_MB = 40
_NB = _CPW // _MB
_EPAD = _NW * _CPW * _K
_RPS = 624
_RTAIL = _N - _RPS * _NS


def _layernorm(x):
    def body(x_ref, o_ref):
        v = x_ref[...]
        m = jnp.mean(v, axis=-1, keepdims=True)
        d = v - m
        var = jnp.mean(d * d, axis=-1, keepdims=True)
        o_ref[...] = d * lax.rsqrt(var + 1e-5)

    return pl.pallas_call(
        body,
        out_shape=jax.ShapeDtypeStruct((_N, _D), jnp.float32),
    )(x)


def _add2(a, b):
    def body(a_ref, b_ref, o_ref):
        o_ref[...] = a_ref[...] + b_ref[...]

    return pl.pallas_call(
        body,
        out_shape=jax.ShapeDtypeStruct((_N, _D), jnp.float32),
    )(a, b)


def _spmm_body(src_h, dst_h, w_h, x_h, init0_h, init1_h, out_h,
               src_v, dst_v, w_v, rows_a, rows_b, acc, sem_a, sem_b):
    c = lax.axis_index("c")
    s = lax.axis_index("s")
    wid = c * _NS + s
    r0 = s * _RPS
    @pl.when(c == 0)
    def _():
        pltpu.sync_copy(init0_h.at[pl.ds(r0, _RPS)], acc.at[pl.ds(r0, _RPS)])

        @pl.when(s == 0)
        def _():
            pltpu.sync_copy(init0_h.at[pl.ds(_RPS * _NS, _RTAIL)],
                            acc.at[pl.ds(_RPS * _NS, _RTAIL)])

    @pl.when(c == 1)
    def _():
        pltpu.sync_copy(init1_h.at[pl.ds(r0, _RPS)], acc.at[pl.ds(r0, _RPS)])

        @pl.when(s == 0)
        def _():
            pltpu.sync_copy(init1_h.at[pl.ds(_RPS * _NS, _RTAIL)],
                            acc.at[pl.ds(_RPS * _NS, _RTAIL)])

    plsc.subcore_barrier()

    def gather_start(g, rows, sem):
        pltpu.async_copy(x_h.at[src_v.at[g]], rows, sem)

    def gather_wait(g, rows, sem):
        pltpu.make_async_copy(x_h.at[src_v.at[g]], rows, sem).wait()

    def scale_scatter(g, rows):
        def scale_body(g16, carry):
            wg = w_v[g, pl.ds(g16 * 16, 16)]
            for e16 in range(16):
                e = g16 * 16 + e16
                wb = wg.at[jnp.full((16,), e16, jnp.int32)].get(
                    mode="promise_in_bounds")
                for r in range(_D // 16):
                    rows[e, pl.ds(r * 16, 16)] = rows[e, pl.ds(r * 16, 16)] * wb
            return carry

        lax.fori_loop(0, _K // 16, scale_body, 0)
        pltpu.sync_copy(rows, acc.at[dst_v.at[g]], add=True)

    def blk_body(b, carry):
        c0 = b * _MB
        pltpu.sync_copy(src_h.at[wid, pl.ds(c0, _MB)], src_v)
        pltpu.sync_copy(dst_h.at[wid, pl.ds(c0, _MB)], dst_v)
        pltpu.sync_copy(w_h.at[wid, pl.ds(c0, _MB)], w_v)
        gather_start(0, rows_a, sem_a)

        def pair_body(i, carry2):
            g = i * 2
            gather_wait(g, rows_a, sem_a)
            gather_start(g + 1, rows_b, sem_b)
            scale_scatter(g, rows_a)

            @pl.when(g + 2 < _MB)
            def _():
                gather_start(g + 2, rows_a, sem_a)

            gather_wait(g + 1, rows_b, sem_b)
            scale_scatter(g + 1, rows_b)
            return carry2

        lax.fori_loop(0, _MB // 2, pair_body, 0)
        return carry

    lax.fori_loop(0, _NB, blk_body, 0)
    plsc.subcore_barrier()
    pltpu.sync_copy(acc.at[pl.ds(r0, _RPS)], out_h.at[c, pl.ds(r0, _RPS)])

    @pl.when(s == 0)
    def _():
        pltpu.sync_copy(acc.at[pl.ds(_RPS * _NS, _RTAIL)],
                        out_h.at[c, pl.ds(_RPS * _NS, _RTAIL)])


def _spmm_sc(src, dst, w, x, init0, init1):
    mesh = plsc.VectorSubcoreMesh(core_axis_name="c", subcore_axis_name="s")
    f = functools.partial(
        pl.kernel,
        out_type=jax.ShapeDtypeStruct((_NC, _N, _D), jnp.float32),
        mesh=mesh,
        scratch_types=[
            pltpu.VMEM((_MB, _K), jnp.int32),
            pltpu.VMEM((_MB, _K), jnp.int32),
            pltpu.VMEM((_MB, _K), jnp.float32),
            pltpu.VMEM((_K, _D), jnp.float32),
            pltpu.VMEM((_K, _D), jnp.float32),
            pltpu.VMEM_SHARED((_N, _D), jnp.float32),
            pltpu.SemaphoreType.DMA,
            pltpu.SemaphoreType.DMA,
        ],
    )(_spmm_body)
    return f(src, dst, w, x, init0, init1)


def kernel(edge_index, edge_weight, embeds):
    src = edge_index[1]
    dst = edge_index[0]
    pad = _EPAD - _E
    spread = (jnp.arange(pad, dtype=jnp.int32) * 13) % _N
    src_p = jnp.concatenate([src, spread]).reshape(_NW, _CPW, _K)
    dst_p = jnp.concatenate([dst, spread]).reshape(_NW, _CPW, _K)
    w_p = jnp.pad(edge_weight, (0, pad)).reshape(_NW, _CPW, _K)

    x = _layernorm(embeds)
    zeros = jnp.zeros((_N, _D), jnp.float32)
    p = _spmm_sc(src_p, dst_p, w_p, x, zeros, zeros)
    y1 = _add2(p[0], p[1])
    q = _spmm_sc(src_p, dst_p, w_p, y1, y1, zeros)
    return _add2(q[0], q[1])
